# baseline (speedup 1.0000x reference)
import functools

import jax
import jax.numpy as jnp
from jax import lax
from jax.experimental import pallas as pl
from jax.experimental.pallas import tpu as pltpu
from jax.experimental.pallas import tpu_sc as plsc

_LOG2E = 1.4426950408889634
_C = 21
_HW = 512 * 512
_B = 8
_NPIX = _B * _HW
_NW = 32
_PIX_W = _NPIX // _NW
_HALF = _PIX_W // 2
_ROWS = _HALF // 128


def _lse_body(y_pred_ref, out_ref):
    b = pl.program_id(0)
    h = pl.program_id(1)
    C, Hb, W = y_pred_ref.shape[1:]
    P = 8

    partial = jnp.zeros((1, W), jnp.float32)
    for p in range(Hb // P):
        rows = pl.ds(p * P, P)
        m = jnp.full((P, W), -jnp.inf, jnp.float32)
        for c in range(C):
            m = jnp.maximum(m, y_pred_ref[0, c, rows, :])
        ml = m * _LOG2E
        s = jnp.zeros((P, W), jnp.float32)
        for c in range(C):
            s += jnp.exp2(y_pred_ref[0, c, rows, :] * _LOG2E - ml)
        partial += jnp.sum(m + jnp.log(s), axis=0, keepdims=True)

    @pl.when((b == 0) & (h == 0))
    def _():
        out_ref[...] = jnp.zeros_like(out_ref)

    out_ref[...] += partial


def _lse_sum(y_pred):
    B, C, H, W = y_pred.shape
    Hb = 128
    out = pl.pallas_call(
        _lse_body,
        grid=(B, H // Hb),
        in_specs=[pl.BlockSpec((1, C, Hb, W), lambda b, h: (b, 0, h, 0))],
        out_specs=pl.BlockSpec((1, W), lambda b, h: (0, 0)),
        out_shape=jax.ShapeDtypeStruct((1, W), jnp.float32),
    )(y_pred)
    return jnp.sum(out)


def _sc_body(ypred_hbm, ytrue_hbm, out_hbm, labels_v, idx_v, vals_v, acc_v, sem):
    nc = 2
    wid = lax.axis_index("s") * nc + lax.axis_index("c")
    base = wid * _PIX_W
    bimg = base // _HW
    add_const = base + bimg * (_C - 1) * _HW

    acc = jnp.zeros((16,), jnp.float32)
    lanes = lax.iota(jnp.int32, 16)

    for half in range(2):
        off = half * _HALF
        pltpu.sync_copy(ytrue_hbm.at[pl.ds(base + off, _HALF)], labels_v)

        def idx_body(r, carry, off=off):
            for sub in range(8):
                k16 = r * 128 + sub * 16
                lv = labels_v[pl.ds(k16, 16)]
                idx_v[pl.ds(k16, 16)] = lv * _HW + (add_const + off + k16) + lanes
            return carry

        lax.fori_loop(0, _ROWS, idx_body, 0)
        pltpu.async_copy(ypred_hbm.at[idx_v], vals_v, sem).wait()

        def sum_body(r, a):
            for sub in range(8):
                a = a + vals_v[pl.ds(r * 128 + sub * 16, 16)]
            return a

        acc = lax.fori_loop(0, _ROWS, sum_body, acc)

    acc_v[...] = acc
    pltpu.sync_copy(acc_v, out_hbm.at[wid])


@functools.partial(
    pl.kernel,
    mesh=plsc.VectorSubcoreMesh(core_axis_name="c", subcore_axis_name="s"),
    out_type=jax.ShapeDtypeStruct((_NW, 16), jnp.float32),
    scratch_types=[
        pltpu.VMEM((_HALF,), jnp.int32),
        pltpu.VMEM((_HALF,), jnp.int32),
        pltpu.VMEM((_HALF,), jnp.float32),
        pltpu.VMEM((16,), jnp.float32),
        pltpu.SemaphoreType.DMA,
    ],
)
def _sc_label_gather_sum(ypred_hbm, ytrue_hbm, out_hbm, *scratch):
    _sc_body(ypred_hbm, ytrue_hbm, out_hbm, *scratch)


def kernel(y_pred, y_true):
    B, C, H, W = y_pred.shape
    lse = _lse_sum(y_pred)
    sel = _sc_label_gather_sum(y_pred.reshape(-1), y_true.reshape(-1))
    return (lse - jnp.sum(sel)) / (B * H * W)


# speedup vs baseline: 3.1495x; 3.1495x over previous
import jax
import jax.numpy as jnp
from jax.experimental import pallas as pl

_LOG2E = 1.4426950408889634


def _ce_body(y_pred_ref, y_true_ref, out_ref):
    b = pl.program_id(0)
    h = pl.program_id(1)
    C, Hb, W = y_pred_ref.shape[1:]
    P = 8

    partial = jnp.zeros((1, W), jnp.float32)
    for p in range(Hb // P):
        rows = pl.ds(p * P, P)
        y = y_true_ref[0, rows, :]
        m = jnp.full((P, W), -jnp.inf, jnp.float32)
        sel = jnp.zeros((P, W), jnp.float32)
        for c in range(C):
            xc = y_pred_ref[0, c, rows, :]
            m = jnp.maximum(m, xc)
            sel += jnp.where(y == c, xc, 0.0)
        ml = m * _LOG2E
        s = jnp.zeros((P, W), jnp.float32)
        for c in range(C):
            xc = y_pred_ref[0, c, rows, :]
            s += jnp.exp2(xc * _LOG2E - ml)
        partial += jnp.sum(m + jnp.log(s) - sel, axis=0, keepdims=True)

    @pl.when((b == 0) & (h == 0))
    def _():
        out_ref[...] = jnp.zeros_like(out_ref)

    out_ref[...] += partial


def kernel(y_pred, y_true):
    B, C, H, W = y_pred.shape
    Hb = 64
    out = pl.pallas_call(
        _ce_body,
        grid=(B, H // Hb),
        in_specs=[
            pl.BlockSpec((1, C, Hb, W), lambda b, h: (b, 0, h, 0)),
            pl.BlockSpec((1, Hb, W), lambda b, h: (b, h, 0)),
        ],
        out_specs=pl.BlockSpec((1, W), lambda b, h: (0, 0)),
        out_shape=jax.ShapeDtypeStruct((1, W), jnp.float32),
    )(y_pred, y_true)
    return jnp.sum(out) / (B * H * W)


# speedup vs baseline: 4.3088x; 1.3681x over previous
import jax
import jax.numpy as jnp
from jax.experimental import pallas as pl

_LOG2E = 1.4426950408889634


def _ce_body(y_pred_ref, y_true_ref, out_ref):
    b = pl.program_id(0)
    h = pl.program_id(1)
    C, Hb, W = y_pred_ref.shape[1:]
    P = 8

    partial = jnp.zeros((1, W), jnp.float32)
    for p in range(Hb // P):
        rows = pl.ds(p * P, P)
        y = y_true_ref[0, rows, :]
        m = jnp.full((P, W), -jnp.inf, jnp.float32)
        sel = jnp.zeros((P, W), jnp.float32)
        for c in range(C):
            xc = y_pred_ref[0, c, rows, :]
            m = jnp.maximum(m, xc)
            sel += jnp.where(y == c, xc, 0.0)
        ml = m * _LOG2E
        s = jnp.zeros((P, W), jnp.float32)
        for c in range(C):
            xc = y_pred_ref[0, c, rows, :]
            s += jnp.exp2(xc * _LOG2E - ml)
        partial += jnp.sum(m + jnp.log(s) - sel, axis=0, keepdims=True)

    @pl.when((b == 0) & (h == 0))
    def _():
        out_ref[...] = jnp.zeros_like(out_ref)

    out_ref[...] += partial


def kernel(y_pred, y_true):
    B, C, H, W = y_pred.shape
    Hb = 256
    out = pl.pallas_call(
        _ce_body,
        grid=(B, H // Hb),
        in_specs=[
            pl.BlockSpec((1, C, Hb, W), lambda b, h: (b, 0, h, 0)),
            pl.BlockSpec((1, Hb, W), lambda b, h: (b, h, 0)),
        ],
        out_specs=pl.BlockSpec((1, W), lambda b, h: (0, 0)),
        out_shape=jax.ShapeDtypeStruct((1, W), jnp.float32),
    )(y_pred, y_true)
    return jnp.sum(out) / (B * H * W)


# speedup vs baseline: 4.4037x; 1.0220x over previous
import jax
import jax.numpy as jnp
from jax.experimental import pallas as pl

_LOG2E = 1.4426950408889634


def _ce_body(y_pred_ref, y_true_ref, out_ref):
    b = pl.program_id(0)
    h = pl.program_id(1)
    C, Hb, W = y_pred_ref.shape[1:]
    P = 8

    partial = jnp.zeros((1, W), jnp.float32)
    for p in range(Hb // P):
        rows = pl.ds(p * P, P)
        y = y_true_ref[0, rows, :]
        m = jnp.full((P, W), -jnp.inf, jnp.float32)
        sel = jnp.zeros((P, W), jnp.float32)
        for c in range(C):
            xc = y_pred_ref[0, c, rows, :]
            m = jnp.maximum(m, xc)
            sel += jnp.where(y == c, xc, 0.0)
        ml = m * _LOG2E
        s = jnp.zeros((P, W), jnp.float32)
        for c in range(C):
            xc = y_pred_ref[0, c, rows, :]
            s += jnp.exp2(xc * _LOG2E - ml)
        partial += jnp.sum(m + jnp.log(s) - sel, axis=0, keepdims=True)

    @pl.when((b == 0) & (h == 0))
    def _():
        out_ref[...] = jnp.zeros_like(out_ref)

    out_ref[...] += partial


def kernel(y_pred, y_true):
    B, C, H, W = y_pred.shape
    Hb = 512
    out = pl.pallas_call(
        _ce_body,
        grid=(B, H // Hb),
        in_specs=[
            pl.BlockSpec((1, C, Hb, W), lambda b, h: (b, 0, h, 0)),
            pl.BlockSpec((1, Hb, W), lambda b, h: (b, h, 0)),
        ],
        out_specs=pl.BlockSpec((1, W), lambda b, h: (0, 0)),
        out_shape=jax.ShapeDtypeStruct((1, W), jnp.float32),
    )(y_pred, y_true)
    return jnp.sum(out) / (B * H * W)
